# Initial kernel scaffold; baseline (speedup 1.0000x reference)
import jax
import jax.numpy as jnp
from jax.experimental import pallas as pl


def kernel(x_user, x_item, W_in_user, b_in_user, W_in_item, b_in_item, miss_user, miss_item, Wl0_ri, bl0_ri, Wr0_ri, Wl0_ru, bl0_ru, Wr0_ru, Wl1_ri, bl1_ri, Wr1_ri, Wl1_ru, bl1_ru, Wr1_ru, Wh1, bh1, Wh2, bh2, ei_rates, ei_rated_by):
    raise NotImplementedError("write your pallas kernel here")


# speedup vs baseline: 6.1306x; 6.1306x over previous
import functools

import jax
import jax.numpy as jnp
from jax import lax
from jax.experimental import pallas as pl
from jax.experimental.pallas import tpu as pltpu
from jax.experimental.pallas import tpu_sc as plsc

N = 50000
E = 800000
H = 64
HH = 32
OUT = 10
NS = 16
LW = 128
ROWS2D = 6272
E_PAD = ROWS2D * LW - E
ROWS_PER_SUB = ROWS2D // NS
CH = 8
NCHUNK = ROWS_PER_SUB // CH
N_PAD = 50048
NODE_SLICE = N_PAD // NS

_mesh = plsc.VectorSubcoreMesh(core_axis_name="c", subcore_axis_name="s")


def _edge_windows(s, sidx, didx, src2d, dst2d, per_window):
    def chunk(k, carry):
        rb = s * ROWS_PER_SUB + k * CH
        pltpu.sync_copy(src2d.at[pl.ds(rb, CH)], sidx)
        pltpu.sync_copy(dst2d.at[pl.ds(rb, CH)], didx)
        for j in range(CH):
            per_window(j)
        return carry

    lax.fori_loop(0, NCHUNK, chunk, 0)


def _sc_mp_body(table_lo, table_hi, src2d, dst2d, zeros,
                out_lo, out_hi, sidx, didx, rows, acc, gsem, ssem):
    c = lax.axis_index("c")
    s = lax.axis_index("s")
    r0 = s * NODE_SLICE
    pltpu.sync_copy(zeros.at[pl.ds(r0, NODE_SLICE)], acc.at[pl.ds(r0, NODE_SLICE)])
    plsc.subcore_barrier()

    def main(table):
        def win(j):
            pltpu.async_copy(table.at[sidx.at[j]], rows, gsem).wait()
            pltpu.async_copy(rows, acc.at[didx.at[j]], ssem, add=True).wait()
        _edge_windows(s, sidx, didx, src2d, dst2d, win)

    @pl.when(c == 0)
    def _():
        main(table_lo)

    @pl.when(c == 1)
    def _():
        main(table_hi)

    plsc.subcore_barrier()

    @pl.when(c == 0)
    def _():
        pltpu.sync_copy(acc.at[pl.ds(r0, NODE_SLICE)], out_lo.at[pl.ds(r0, NODE_SLICE)])

    @pl.when(c == 1)
    def _():
        pltpu.sync_copy(acc.at[pl.ds(r0, NODE_SLICE)], out_hi.at[pl.ds(r0, NODE_SLICE)])


_sc_mp = pl.kernel(
    _sc_mp_body,
    out_type=[jax.ShapeDtypeStruct((N_PAD, HH), jnp.float32)] * 2,
    mesh=_mesh,
    scratch_types=[
        pltpu.VMEM((CH, LW), jnp.int32),
        pltpu.VMEM((CH, LW), jnp.int32),
        pltpu.VMEM((LW, HH), jnp.float32),
        pltpu.VMEM_SHARED((N_PAD, HH), jnp.float32),
        pltpu.SemaphoreType.DMA,
        pltpu.SemaphoreType.DMA,
    ],
    compiler_params=pltpu.CompilerParams(use_tc_tiling_on_sc=False),
)


def _sc_cnt_body(dst2d_a, dst2d_b, ones_hbm, zeros,
                 out_a, out_b, didx, ones_v, acc, osem, ssem):
    c = lax.axis_index("c")
    s = lax.axis_index("s")
    r0 = s * NODE_SLICE
    pltpu.async_copy(ones_hbm, ones_v, osem).wait()
    pltpu.sync_copy(zeros.at[pl.ds(r0, NODE_SLICE)], acc.at[pl.ds(r0, NODE_SLICE)])
    plsc.subcore_barrier()

    def main(dst2d):
        def win(j):
            pltpu.async_copy(ones_v, acc.at[didx.at[j]], ssem, add=True).wait()
        _edge_windows(s, didx, didx, dst2d, dst2d, win)

    @pl.when(c == 0)
    def _():
        main(dst2d_a)

    @pl.when(c == 1)
    def _():
        main(dst2d_b)

    plsc.subcore_barrier()

    @pl.when(c == 0)
    def _():
        pltpu.sync_copy(acc.at[pl.ds(r0, NODE_SLICE)], out_a.at[pl.ds(r0, NODE_SLICE)])

    @pl.when(c == 1)
    def _():
        pltpu.sync_copy(acc.at[pl.ds(r0, NODE_SLICE)], out_b.at[pl.ds(r0, NODE_SLICE)])


_sc_cnt = pl.kernel(
    _sc_cnt_body,
    out_type=[jax.ShapeDtypeStruct((N_PAD, 16), jnp.float32)] * 2,
    mesh=_mesh,
    scratch_types=[
        pltpu.VMEM((CH, LW), jnp.int32),
        pltpu.VMEM((LW, 16), jnp.float32),
        pltpu.VMEM_SHARED((N_PAD, 16), jnp.float32),
        pltpu.SemaphoreType.DMA,
        pltpu.SemaphoreType.DMA,
    ],
    compiler_params=pltpu.CompilerParams(use_tc_tiling_on_sc=False),
)


RB = 2000
GRID = N // RB


def _proj_body(xu_ref, xi_ref, wu_ref, bu_ref, wi_ref, bi_ref, mu_ref, mi_ref,
               hu_lo, hu_hi, hi_lo, hi_hi):
    def proj(x, w, b, m):
        h = (x[:, 0:1] * w[0:1, :] + x[:, 1:2] * (w[1:2, :] + m[...])
             + b[...])
        return jnp.maximum(h, 0.0)

    hu = proj(xu_ref[...], wu_ref, bu_ref, mu_ref)
    hi = proj(xi_ref[...], wi_ref, bi_ref, mi_ref)
    hu_lo[...] = hu[:, :HH]
    hu_hi[...] = hu[:, HH:]
    hi_lo[...] = hi[:, :HH]
    hi_hi[...] = hi[:, HH:]


def _tc_proj(x_user, x_item, W_u, b_u, W_i, b_i, miss_u, miss_i):
    row = pl.BlockSpec((RB, 2), lambda i: (i, 0))
    full = lambda shp: pl.BlockSpec(shp, lambda i: (0, 0))
    oblk = pl.BlockSpec((RB, HH), lambda i: (i, 0))
    return pl.pallas_call(
        _proj_body,
        grid=(GRID,),
        in_specs=[row, row, full((2, H)), full((1, H)), full((2, H)),
                  full((1, H)), full((1, H)), full((1, H))],
        out_specs=[oblk] * 4,
        out_shape=[jax.ShapeDtypeStruct((N_PAD, HH), jnp.float32)] * 4,
    )(x_user, x_item, W_u, b_u.reshape(1, H), W_i, b_i.reshape(1, H),
      miss_u, miss_i)


def _sage_out(agg_lo, agg_hi, cnt, h_lo, h_hi, wl, bl, wr):
    agg = jnp.concatenate([agg_lo[...], agg_hi[...]], axis=1)
    h = jnp.concatenate([h_lo[...], h_hi[...]], axis=1)
    mean = agg / jnp.maximum(cnt[:, 0:1], 1.0)
    return jnp.maximum(mean @ wl[...] + bl[...] + h @ wr[...], 0.0)


def _layer_body(ai_lo, ai_hi, ci, hi_lo, hi_hi, wli, bli, wri,
                au_lo, au_hi, cu, hu_lo, hu_hi, wlu, blu, wru,
                ni_lo, ni_hi, nu_lo, nu_hi):
    oi = _sage_out(ai_lo, ai_hi, ci, hi_lo, hi_hi, wli, bli, wri)
    ou = _sage_out(au_lo, au_hi, cu, hu_lo, hu_hi, wlu, blu, wru)
    ni_lo[...] = oi[:, :HH]
    ni_hi[...] = oi[:, HH:]
    nu_lo[...] = ou[:, :HH]
    nu_hi[...] = ou[:, HH:]


def _final_body(ai_lo, ai_hi, ci, hi_lo, hi_hi, wli, bli, wri,
                au_lo, au_hi, cu, hu_lo, hu_hi, wlu, blu, wru,
                sum_u, sum_i):
    i = pl.program_id(0)
    oi = _sage_out(ai_lo, ai_hi, ci, hi_lo, hi_hi, wli, bli, wri)
    ou = _sage_out(au_lo, au_hi, cu, hu_lo, hu_hi, wlu, blu, wru)

    @pl.when(i == 0)
    def _():
        sum_u[...] = jnp.zeros_like(sum_u)
        sum_i[...] = jnp.zeros_like(sum_i)

    sum_u[...] += jnp.sum(ou, axis=0, keepdims=True)
    sum_i[...] += jnp.sum(oi, axis=0, keepdims=True)


def _layer_specs():
    blk = pl.BlockSpec((RB, HH), lambda i: (i, 0))
    cblk = pl.BlockSpec((RB, 16), lambda i: (i, 0))
    w = pl.BlockSpec((H, H), lambda i: (0, 0))
    b = pl.BlockSpec((1, H), lambda i: (0, 0))
    return [blk, blk, cblk, blk, blk, w, b, w,
            blk, blk, cblk, blk, blk, w, b, w]


def _tc_layer(ai, ci, hi, wli, bli, wri, au, cu, hu, wlu, blu, wru):
    oblk = pl.BlockSpec((RB, HH), lambda i: (i, 0))
    return pl.pallas_call(
        _layer_body,
        grid=(GRID,),
        in_specs=_layer_specs(),
        out_specs=[oblk] * 4,
        out_shape=[jax.ShapeDtypeStruct((N_PAD, HH), jnp.float32)] * 4,
    )(ai[0], ai[1], ci, hi[0], hi[1], wli, bli.reshape(1, H), wri,
      au[0], au[1], cu, hu[0], hu[1], wlu, blu.reshape(1, H), wru)


def _tc_final(ai, ci, hi, wli, bli, wri, au, cu, hu, wlu, blu, wru):
    sblk = pl.BlockSpec((1, H), lambda i: (0, 0))
    return pl.pallas_call(
        _final_body,
        grid=(GRID,),
        in_specs=_layer_specs(),
        out_specs=[sblk, sblk],
        out_shape=[jax.ShapeDtypeStruct((1, H), jnp.float32)] * 2,
    )(ai[0], ai[1], ci, hi[0], hi[1], wli, bli.reshape(1, H), wri,
      au[0], au[1], cu, hu[0], hu[1], wlu, blu.reshape(1, H), wru)


def _head_body(su, si, w1, b1, w2, b2, out):
    pooled = jnp.concatenate([su[...], si[...]], axis=1)
    hid = jnp.maximum(pooled @ w1[...] + b1[...], 0.0)
    out[...] = hid @ w2[...] + b2[...]


def _tc_head(sum_u, sum_i, Wh1, bh1, Wh2, bh2):
    return pl.pallas_call(
        _head_body,
        out_shape=jax.ShapeDtypeStruct((1, OUT), jnp.float32),
    )(sum_u, sum_i, Wh1, bh1.reshape(1, H), Wh2, bh2.reshape(1, OUT))


def kernel(x_user, x_item, W_in_user, b_in_user, W_in_item, b_in_item,
           miss_user, miss_item,
           Wl0_ri, bl0_ri, Wr0_ri, Wl0_ru, bl0_ru, Wr0_ru,
           Wl1_ri, bl1_ri, Wr1_ri, Wl1_ru, bl1_ru, Wr1_ru,
           Wh1, bh1, Wh2, bh2,
           ei_rates, ei_rated_by):
    pad = (N + (jnp.arange(E_PAD, dtype=jnp.int32) % (N_PAD - N))).astype(jnp.int32)

    def edges(ei):
        return (jnp.concatenate([ei[0], pad]).reshape(ROWS2D, LW),
                jnp.concatenate([ei[1], pad]).reshape(ROWS2D, LW))

    src_ri, dst_ri = edges(ei_rates)
    src_ru, dst_ru = edges(ei_rated_by)

    z32 = jnp.zeros((N_PAD, HH), jnp.float32)
    z16 = jnp.zeros((N_PAD, 16), jnp.float32)
    ones16 = jnp.ones((LW, 16), jnp.float32)

    hu = _tc_proj(x_user, x_item, W_in_user, b_in_user, W_in_item, b_in_item,
                  miss_user, miss_item)
    h_u, h_i = (hu[0], hu[1]), (hu[2], hu[3])

    cnt_i, cnt_u = _sc_cnt(dst_ri, dst_ru, ones16, z16)

    agg_i = _sc_mp(h_u[0], h_u[1], src_ri, dst_ri, z32)
    agg_u = _sc_mp(h_i[0], h_i[1], src_ru, dst_ru, z32)
    nh = _tc_layer(agg_i, cnt_i, h_i, Wl0_ri, bl0_ri, Wr0_ri,
                   agg_u, cnt_u, h_u, Wl0_ru, bl0_ru, Wr0_ru)
    h_i, h_u = (nh[0], nh[1]), (nh[2], nh[3])

    agg_i = _sc_mp(h_u[0], h_u[1], src_ri, dst_ri, z32)
    agg_u = _sc_mp(h_i[0], h_i[1], src_ru, dst_ru, z32)
    sum_u, sum_i = _tc_final(agg_i, cnt_i, h_i, Wl1_ri, bl1_ri, Wr1_ri,
                             agg_u, cnt_u, h_u, Wl1_ru, bl1_ru, Wr1_ru)

    return _tc_head(sum_u, sum_i, Wh1, bh1, Wh2, bh2)


# speedup vs baseline: 11.5663x; 1.8867x over previous
import functools

import jax
import jax.numpy as jnp
from jax import lax
from jax.experimental import pallas as pl
from jax.experimental.pallas import tpu as pltpu
from jax.experimental.pallas import tpu_sc as plsc

N = 50000
E = 800000
H = 64
HH = 32
OUT = 10
NS = 16
LW = 128
ROWS2D = 6272
E_PAD = ROWS2D * LW - E
ROWS_PER_SUB = ROWS2D // NS
CH = 4
NCHUNK = ROWS_PER_SUB // CH
N_PAD = 50048
NODE_SLICE = N_PAD // NS

_mesh = plsc.VectorSubcoreMesh(core_axis_name="c", subcore_axis_name="s")


def _edge_windows(s, sidx, didx, src2d, dst2d, per_window):
    def chunk(k, carry):
        rb = s * ROWS_PER_SUB + k * CH
        pltpu.sync_copy(src2d.at[pl.ds(rb, CH)], sidx)
        pltpu.sync_copy(dst2d.at[pl.ds(rb, CH)], didx)
        for j in range(CH):
            per_window(j)
        return carry

    lax.fori_loop(0, NCHUNK, chunk, 0)


def _sc_mp_body(table_lo, table_hi, src2d, dst2d, zeros,
                out_lo, out_hi, sidxA, didxA, sidxB, didxB, rbuf, acc,
                gsem, ssem, isem):
    c = lax.axis_index("c")
    s = lax.axis_index("s")
    r0 = s * NODE_SLICE
    pltpu.sync_copy(zeros.at[pl.ds(r0, NODE_SLICE)], acc.at[pl.ds(r0, NODE_SLICE)])
    plsc.subcore_barrier()

    def main(table):
        def stage(k, sp, dp, sem):
            rb = s * ROWS_PER_SUB + k * CH
            pltpu.async_copy(src2d.at[pl.ds(rb, CH)], sp, sem)
            pltpu.async_copy(dst2d.at[pl.ds(rb, CH)], dp, sem)

        def process(k, par, is_first):
            sp, dp = (sidxA, didxA) if par == 0 else (sidxB, didxB)
            nsp, ndp = (sidxB, didxB) if par == 0 else (sidxA, didxA)

            if not is_first:
                pltpu.make_async_copy(src2d.at[pl.ds(0, CH)], sp,
                                      isem.at[par]).wait()
                pltpu.make_async_copy(dst2d.at[pl.ds(0, CH)], dp,
                                      isem.at[par]).wait()
            for j in range(CH):
                if not is_first:
                    pltpu.make_async_copy(rbuf.at[j], acc.at[dp.at[j]],
                                          ssem.at[j]).wait()
                pltpu.async_copy(table.at[sp.at[j]], rbuf.at[j], gsem.at[j])

            @pl.when(k + 1 < NCHUNK)
            def _():
                stage(k + 1, nsp, ndp, isem.at[1 - par])

            for j in range(CH):
                pltpu.make_async_copy(table.at[sp.at[j]], rbuf.at[j],
                                      gsem.at[j]).wait()
                pltpu.async_copy(rbuf.at[j], acc.at[dp.at[j]], ssem.at[j],
                                 add=True)

        stage(0, sidxA, didxA, isem.at[0])
        pltpu.make_async_copy(src2d.at[pl.ds(0, CH)], sidxA, isem.at[0]).wait()
        pltpu.make_async_copy(dst2d.at[pl.ds(0, CH)], didxA, isem.at[0]).wait()
        process(0, 0, True)

        def pair(g, carry):
            process(2 * g + 1, 1, False)
            process(2 * g + 2, 0, False)
            return carry

        lax.fori_loop(0, (NCHUNK - 1) // 2, pair, 0)
        if (NCHUNK - 1) % 2 == 1:
            process(NCHUNK - 1, (NCHUNK - 1) % 2, False)
        last_d = didxA if (NCHUNK - 1) % 2 == 0 else didxB
        for j in range(CH):
            pltpu.make_async_copy(rbuf.at[j], acc.at[last_d.at[j]],
                                  ssem.at[j]).wait()

    @pl.when(c == 0)
    def _():
        main(table_lo)

    @pl.when(c == 1)
    def _():
        main(table_hi)

    plsc.subcore_barrier()

    @pl.when(c == 0)
    def _():
        pltpu.sync_copy(acc.at[pl.ds(r0, NODE_SLICE)], out_lo.at[pl.ds(r0, NODE_SLICE)])

    @pl.when(c == 1)
    def _():
        pltpu.sync_copy(acc.at[pl.ds(r0, NODE_SLICE)], out_hi.at[pl.ds(r0, NODE_SLICE)])


_sc_mp = pl.kernel(
    _sc_mp_body,
    out_type=[jax.ShapeDtypeStruct((N_PAD, HH), jnp.float32)] * 2,
    mesh=_mesh,
    scratch_types=[
        pltpu.VMEM((CH, LW), jnp.int32),
        pltpu.VMEM((CH, LW), jnp.int32),
        pltpu.VMEM((CH, LW), jnp.int32),
        pltpu.VMEM((CH, LW), jnp.int32),
        pltpu.VMEM((CH, LW, HH), jnp.float32),
        pltpu.VMEM_SHARED((N_PAD, HH), jnp.float32),
        pltpu.SemaphoreType.DMA((CH,)),
        pltpu.SemaphoreType.DMA((CH,)),
        pltpu.SemaphoreType.DMA((2,)),
    ],
    compiler_params=pltpu.CompilerParams(use_tc_tiling_on_sc=False),
)


def _sc_cnt_body(dst2d_a, dst2d_b, ones_hbm, zeros,
                 out_a, out_b, didx, ones_v, acc, osem, ssem):
    c = lax.axis_index("c")
    s = lax.axis_index("s")
    r0 = s * NODE_SLICE
    pltpu.async_copy(ones_hbm, ones_v, osem).wait()
    pltpu.sync_copy(zeros.at[pl.ds(r0, NODE_SLICE)], acc.at[pl.ds(r0, NODE_SLICE)])
    plsc.subcore_barrier()

    def main(dst2d):
        def win(j):
            pltpu.async_copy(ones_v, acc.at[didx.at[j]], ssem, add=True).wait()
        _edge_windows(s, didx, didx, dst2d, dst2d, win)

    @pl.when(c == 0)
    def _():
        main(dst2d_a)

    @pl.when(c == 1)
    def _():
        main(dst2d_b)

    plsc.subcore_barrier()

    @pl.when(c == 0)
    def _():
        pltpu.sync_copy(acc.at[pl.ds(r0, NODE_SLICE)], out_a.at[pl.ds(r0, NODE_SLICE)])

    @pl.when(c == 1)
    def _():
        pltpu.sync_copy(acc.at[pl.ds(r0, NODE_SLICE)], out_b.at[pl.ds(r0, NODE_SLICE)])


_sc_cnt = pl.kernel(
    _sc_cnt_body,
    out_type=[jax.ShapeDtypeStruct((N_PAD, 16), jnp.float32)] * 2,
    mesh=_mesh,
    scratch_types=[
        pltpu.VMEM((CH, LW), jnp.int32),
        pltpu.VMEM((LW, 16), jnp.float32),
        pltpu.VMEM_SHARED((N_PAD, 16), jnp.float32),
        pltpu.SemaphoreType.DMA,
        pltpu.SemaphoreType.DMA,
    ],
    compiler_params=pltpu.CompilerParams(use_tc_tiling_on_sc=False),
)


RB = 2000
GRID = N // RB


def _proj_body(xu_ref, xi_ref, wu_ref, bu_ref, wi_ref, bi_ref, mu_ref, mi_ref,
               hu_lo, hu_hi, hi_lo, hi_hi):
    def proj(x, w, b, m):
        h = (x[:, 0:1] * w[0:1, :] + x[:, 1:2] * (w[1:2, :] + m[...])
             + b[...])
        return jnp.maximum(h, 0.0)

    hu = proj(xu_ref[...], wu_ref, bu_ref, mu_ref)
    hi = proj(xi_ref[...], wi_ref, bi_ref, mi_ref)
    hu_lo[...] = hu[:, :HH]
    hu_hi[...] = hu[:, HH:]
    hi_lo[...] = hi[:, :HH]
    hi_hi[...] = hi[:, HH:]


def _tc_proj(x_user, x_item, W_u, b_u, W_i, b_i, miss_u, miss_i):
    row = pl.BlockSpec((RB, 2), lambda i: (i, 0))
    full = lambda shp: pl.BlockSpec(shp, lambda i: (0, 0))
    oblk = pl.BlockSpec((RB, HH), lambda i: (i, 0))
    return pl.pallas_call(
        _proj_body,
        grid=(GRID,),
        in_specs=[row, row, full((2, H)), full((1, H)), full((2, H)),
                  full((1, H)), full((1, H)), full((1, H))],
        out_specs=[oblk] * 4,
        out_shape=[jax.ShapeDtypeStruct((N_PAD, HH), jnp.float32)] * 4,
    )(x_user, x_item, W_u, b_u.reshape(1, H), W_i, b_i.reshape(1, H),
      miss_u, miss_i)


def _sage_out(agg_lo, agg_hi, cnt, h_lo, h_hi, wl, bl, wr):
    agg = jnp.concatenate([agg_lo[...], agg_hi[...]], axis=1)
    h = jnp.concatenate([h_lo[...], h_hi[...]], axis=1)
    mean = agg / jnp.maximum(cnt[:, 0:1], 1.0)
    return jnp.maximum(mean @ wl[...] + bl[...] + h @ wr[...], 0.0)


def _layer_body(ai_lo, ai_hi, ci, hi_lo, hi_hi, wli, bli, wri,
                au_lo, au_hi, cu, hu_lo, hu_hi, wlu, blu, wru,
                ni_lo, ni_hi, nu_lo, nu_hi):
    oi = _sage_out(ai_lo, ai_hi, ci, hi_lo, hi_hi, wli, bli, wri)
    ou = _sage_out(au_lo, au_hi, cu, hu_lo, hu_hi, wlu, blu, wru)
    ni_lo[...] = oi[:, :HH]
    ni_hi[...] = oi[:, HH:]
    nu_lo[...] = ou[:, :HH]
    nu_hi[...] = ou[:, HH:]


def _final_body(ai_lo, ai_hi, ci, hi_lo, hi_hi, wli, bli, wri,
                au_lo, au_hi, cu, hu_lo, hu_hi, wlu, blu, wru,
                sum_u, sum_i):
    i = pl.program_id(0)
    oi = _sage_out(ai_lo, ai_hi, ci, hi_lo, hi_hi, wli, bli, wri)
    ou = _sage_out(au_lo, au_hi, cu, hu_lo, hu_hi, wlu, blu, wru)

    @pl.when(i == 0)
    def _():
        sum_u[...] = jnp.zeros_like(sum_u)
        sum_i[...] = jnp.zeros_like(sum_i)

    sum_u[...] += jnp.sum(ou, axis=0, keepdims=True)
    sum_i[...] += jnp.sum(oi, axis=0, keepdims=True)


def _layer_specs():
    blk = pl.BlockSpec((RB, HH), lambda i: (i, 0))
    cblk = pl.BlockSpec((RB, 16), lambda i: (i, 0))
    w = pl.BlockSpec((H, H), lambda i: (0, 0))
    b = pl.BlockSpec((1, H), lambda i: (0, 0))
    return [blk, blk, cblk, blk, blk, w, b, w,
            blk, blk, cblk, blk, blk, w, b, w]


def _tc_layer(ai, ci, hi, wli, bli, wri, au, cu, hu, wlu, blu, wru):
    oblk = pl.BlockSpec((RB, HH), lambda i: (i, 0))
    return pl.pallas_call(
        _layer_body,
        grid=(GRID,),
        in_specs=_layer_specs(),
        out_specs=[oblk] * 4,
        out_shape=[jax.ShapeDtypeStruct((N_PAD, HH), jnp.float32)] * 4,
    )(ai[0], ai[1], ci, hi[0], hi[1], wli, bli.reshape(1, H), wri,
      au[0], au[1], cu, hu[0], hu[1], wlu, blu.reshape(1, H), wru)


def _tc_final(ai, ci, hi, wli, bli, wri, au, cu, hu, wlu, blu, wru):
    sblk = pl.BlockSpec((1, H), lambda i: (0, 0))
    return pl.pallas_call(
        _final_body,
        grid=(GRID,),
        in_specs=_layer_specs(),
        out_specs=[sblk, sblk],
        out_shape=[jax.ShapeDtypeStruct((1, H), jnp.float32)] * 2,
    )(ai[0], ai[1], ci, hi[0], hi[1], wli, bli.reshape(1, H), wri,
      au[0], au[1], cu, hu[0], hu[1], wlu, blu.reshape(1, H), wru)


def _head_body(su, si, w1, b1, w2, b2, out):
    pooled = jnp.concatenate([su[...], si[...]], axis=1)
    hid = jnp.maximum(pooled @ w1[...] + b1[...], 0.0)
    out[...] = hid @ w2[...] + b2[...]


def _tc_head(sum_u, sum_i, Wh1, bh1, Wh2, bh2):
    return pl.pallas_call(
        _head_body,
        out_shape=jax.ShapeDtypeStruct((1, OUT), jnp.float32),
    )(sum_u, sum_i, Wh1, bh1.reshape(1, H), Wh2, bh2.reshape(1, OUT))


def kernel(x_user, x_item, W_in_user, b_in_user, W_in_item, b_in_item,
           miss_user, miss_item,
           Wl0_ri, bl0_ri, Wr0_ri, Wl0_ru, bl0_ru, Wr0_ru,
           Wl1_ri, bl1_ri, Wr1_ri, Wl1_ru, bl1_ru, Wr1_ru,
           Wh1, bh1, Wh2, bh2,
           ei_rates, ei_rated_by):
    pad = (N + (jnp.arange(E_PAD, dtype=jnp.int32) % (N_PAD - N))).astype(jnp.int32)

    def edges(ei):
        return (jnp.concatenate([ei[0], pad]).reshape(ROWS2D, LW),
                jnp.concatenate([ei[1], pad]).reshape(ROWS2D, LW))

    src_ri, dst_ri = edges(ei_rates)
    src_ru, dst_ru = edges(ei_rated_by)

    z32 = jnp.zeros((N_PAD, HH), jnp.float32)
    z16 = jnp.zeros((N_PAD, 16), jnp.float32)
    ones16 = jnp.ones((LW, 16), jnp.float32)

    hu = _tc_proj(x_user, x_item, W_in_user, b_in_user, W_in_item, b_in_item,
                  miss_user, miss_item)
    h_u, h_i = (hu[0], hu[1]), (hu[2], hu[3])

    cnt_i, cnt_u = _sc_cnt(dst_ri, dst_ru, ones16, z16)

    agg_i = _sc_mp(h_u[0], h_u[1], src_ri, dst_ri, z32)
    agg_u = _sc_mp(h_i[0], h_i[1], src_ru, dst_ru, z32)
    nh = _tc_layer(agg_i, cnt_i, h_i, Wl0_ri, bl0_ri, Wr0_ri,
                   agg_u, cnt_u, h_u, Wl0_ru, bl0_ru, Wr0_ru)
    h_i, h_u = (nh[0], nh[1]), (nh[2], nh[3])

    agg_i = _sc_mp(h_u[0], h_u[1], src_ri, dst_ri, z32)
    agg_u = _sc_mp(h_i[0], h_i[1], src_ru, dst_ru, z32)
    sum_u, sum_i = _tc_final(agg_i, cnt_i, h_i, Wl1_ri, bl1_ri, Wr1_ri,
                             agg_u, cnt_u, h_u, Wl1_ru, bl1_ru, Wr1_ru)

    return _tc_head(sum_u, sum_i, Wh1, bh1, Wh2, bh2)


# speedup vs baseline: 17.1780x; 1.4852x over previous
import jax
import jax.numpy as jnp
from jax import lax
from jax.experimental import pallas as pl
from jax.experimental.pallas import tpu as pltpu
from jax.experimental.pallas import tpu_sc as plsc

N = 50000
E = 800000
H = 64
HH = 32
OUT = 10
NS = 16
LW = 128
ROWS2D = 6272
E_PAD = ROWS2D * LW - E
ROWS_PER_SUB = ROWS2D // NS
CH = 4
NCHUNK = ROWS_PER_SUB // CH
N_PAD = 50048
NODE_SLICE = N_PAD // NS
NP4 = N_PAD // 4

_mesh = plsc.VectorSubcoreMesh(core_axis_name="c", subcore_axis_name="s")


def _sc_mp_body(table_lo, table_hi, src2d, dst2d, zeros,
                out_lo, out_hi, sidxA, didxA, sidxB, didxB, rbuf, acc,
                gsem, ssem, isem):
    c = lax.axis_index("c")
    s = lax.axis_index("s")
    r0 = s * NODE_SLICE
    pltpu.sync_copy(zeros.at[pl.ds(r0, NODE_SLICE)], acc.at[pl.ds(r0, NODE_SLICE)])
    plsc.subcore_barrier()

    def main(table):
        def stage(k, sp, dp, sem):
            rb = s * ROWS_PER_SUB + k * CH
            pltpu.async_copy(src2d.at[pl.ds(rb, CH)], sp, sem)
            pltpu.async_copy(dst2d.at[pl.ds(rb, CH)], dp, sem)

        def process(k, par, is_first):
            sp, dp = (sidxA, didxA) if par == 0 else (sidxB, didxB)
            nsp, ndp = (sidxB, didxB) if par == 0 else (sidxA, didxA)

            if not is_first:
                pltpu.make_async_copy(src2d.at[pl.ds(0, CH)], sp,
                                      isem.at[par]).wait()
                pltpu.make_async_copy(dst2d.at[pl.ds(0, CH)], dp,
                                      isem.at[par]).wait()
            for j in range(CH):
                if not is_first:
                    pltpu.make_async_copy(rbuf.at[j], acc.at[dp.at[j]],
                                          ssem.at[j]).wait()
                pltpu.async_copy(table.at[sp.at[j]], rbuf.at[j], gsem.at[j])

            @pl.when(k + 1 < NCHUNK)
            def _():
                stage(k + 1, nsp, ndp, isem.at[1 - par])

            for j in range(CH):
                pltpu.make_async_copy(table.at[sp.at[j]], rbuf.at[j],
                                      gsem.at[j]).wait()
                pltpu.async_copy(rbuf.at[j], acc.at[dp.at[j]], ssem.at[j],
                                 add=True)

        stage(0, sidxA, didxA, isem.at[0])
        pltpu.make_async_copy(src2d.at[pl.ds(0, CH)], sidxA, isem.at[0]).wait()
        pltpu.make_async_copy(dst2d.at[pl.ds(0, CH)], didxA, isem.at[0]).wait()
        process(0, 0, True)

        def pair(g, carry):
            process(2 * g + 1, 1, False)
            process(2 * g + 2, 0, False)
            return carry

        lax.fori_loop(0, (NCHUNK - 1) // 2, pair, 0)
        if (NCHUNK - 1) % 2 == 1:
            process(NCHUNK - 1, (NCHUNK - 1) % 2, False)
        last_d = didxA if (NCHUNK - 1) % 2 == 0 else didxB
        for j in range(CH):
            pltpu.make_async_copy(rbuf.at[j], acc.at[last_d.at[j]],
                                  ssem.at[j]).wait()

    @pl.when(c == 0)
    def _():
        main(table_lo)

    @pl.when(c == 1)
    def _():
        main(table_hi)

    plsc.subcore_barrier()

    @pl.when(c == 0)
    def _():
        pltpu.sync_copy(acc.at[pl.ds(r0, NODE_SLICE)], out_lo.at[pl.ds(r0, NODE_SLICE)])

    @pl.when(c == 1)
    def _():
        pltpu.sync_copy(acc.at[pl.ds(r0, NODE_SLICE)], out_hi.at[pl.ds(r0, NODE_SLICE)])


_sc_mp = pl.kernel(
    _sc_mp_body,
    out_type=[jax.ShapeDtypeStruct((N_PAD, HH), jnp.float32)] * 2,
    mesh=_mesh,
    scratch_types=[
        pltpu.VMEM((CH, LW), jnp.int32),
        pltpu.VMEM((CH, LW), jnp.int32),
        pltpu.VMEM((CH, LW), jnp.int32),
        pltpu.VMEM((CH, LW), jnp.int32),
        pltpu.VMEM((CH, LW, HH), jnp.float32),
        pltpu.VMEM_SHARED((N_PAD, HH), jnp.float32),
        pltpu.SemaphoreType.DMA((CH,)),
        pltpu.SemaphoreType.DMA((CH,)),
        pltpu.SemaphoreType.DMA((2,)),
    ],
    compiler_params=pltpu.CompilerParams(use_tc_tiling_on_sc=False),
)


def _sc_cnt_body(dst2d_a, dst2d_b, ones_hbm, zeros,
                 out_a, out_b, didxA, didxB, ones_v, acc, osem, ssem, isem):
    c = lax.axis_index("c")
    s = lax.axis_index("s")
    r0 = s * NODE_SLICE
    pltpu.async_copy(ones_hbm, ones_v, osem).wait()
    pltpu.sync_copy(zeros.at[pl.ds(r0, NODE_SLICE)], acc.at[pl.ds(r0, NODE_SLICE)])
    plsc.subcore_barrier()

    def main(dst2d):
        def stage(k, dp, sem):
            rb = s * ROWS_PER_SUB + k * CH
            pltpu.async_copy(dst2d.at[pl.ds(rb, CH)], dp, sem)

        def process(k, par, is_first):
            dp = didxA if par == 0 else didxB
            ndp = didxB if par == 0 else didxA
            if not is_first:
                pltpu.make_async_copy(dst2d.at[pl.ds(0, CH)], dp,
                                      isem.at[par]).wait()
            for j in range(CH):
                pltpu.async_copy(ones_v, acc.at[dp.at[j]],
                                 ssem.at[par * CH + j], add=True)
            if not is_first:
                for j in range(CH):
                    pltpu.make_async_copy(ones_v, acc.at[ndp.at[j]],
                                          ssem.at[(1 - par) * CH + j]).wait()

            @pl.when(k + 1 < NCHUNK)
            def _():
                stage(k + 1, ndp, isem.at[1 - par])

        stage(0, didxA, isem.at[0])
        pltpu.make_async_copy(dst2d.at[pl.ds(0, CH)], didxA, isem.at[0]).wait()
        process(0, 0, True)

        def pair(g, carry):
            process(2 * g + 1, 1, False)
            process(2 * g + 2, 0, False)
            return carry

        lax.fori_loop(0, (NCHUNK - 1) // 2, pair, 0)
        if (NCHUNK - 1) % 2 == 1:
            process(NCHUNK - 1, (NCHUNK - 1) % 2, False)
        lpar = (NCHUNK - 1) % 2
        ld = didxA if lpar == 0 else didxB
        for j in range(CH):
            pltpu.make_async_copy(ones_v, acc.at[ld.at[j]],
                                  ssem.at[lpar * CH + j]).wait()

    @pl.when(c == 0)
    def _():
        main(dst2d_a)

    @pl.when(c == 1)
    def _():
        main(dst2d_b)

    plsc.subcore_barrier()

    @pl.when(c == 0)
    def _():
        pltpu.sync_copy(acc.at[pl.ds(r0, NODE_SLICE)], out_a.at[pl.ds(r0, NODE_SLICE)])

    @pl.when(c == 1)
    def _():
        pltpu.sync_copy(acc.at[pl.ds(r0, NODE_SLICE)], out_b.at[pl.ds(r0, NODE_SLICE)])


_sc_cnt = pl.kernel(
    _sc_cnt_body,
    out_type=[jax.ShapeDtypeStruct((N_PAD, HH), jnp.float32)] * 2,
    mesh=_mesh,
    scratch_types=[
        pltpu.VMEM((CH, LW), jnp.int32),
        pltpu.VMEM((CH, LW), jnp.int32),
        pltpu.VMEM((LW, HH), jnp.float32),
        pltpu.VMEM_SHARED((N_PAD, HH), jnp.float32),
        pltpu.SemaphoreType.DMA,
        pltpu.SemaphoreType.DMA((2 * CH,)),
        pltpu.SemaphoreType.DMA((2,)),
    ],
    compiler_params=pltpu.CompilerParams(use_tc_tiling_on_sc=False),
)


RBP = 512
GRID = 25


def _expand_w(Wfull):
    Wb = Wfull.reshape(2, 32, 2, 32)
    eye4 = jnp.eye(4, dtype=Wfull.dtype)
    return jnp.einsum("ab,pkqj->pakqbj", eye4, Wb).reshape(256, 256)


def _tile_bias(b):
    return jnp.concatenate(
        [jnp.tile(b[:HH], 4), jnp.tile(b[HH:], 4)]).reshape(1, 256)


def _proj_body(xu_ref, xi_ref, bu_ref, btu_ref, bi_ref, bti_ref,
               hu_lo, hu_hi, hi_lo, hi_hi):
    def proj(x, Bw, bt):
        h = jnp.maximum(x[...] @ Bw[...] + bt[...], 0.0)
        return h[:, :128], h[:, 128:]

    hu_lo[...], hu_hi[...] = proj(xu_ref, bu_ref, btu_ref)
    hi_lo[...], hi_hi[...] = proj(xi_ref, bi_ref, bti_ref)


def _tc_proj(xu8, xi8, Bu, btu, Bi, bti):
    xblk = pl.BlockSpec((RBP, 8), lambda i: (i, 0))
    full = lambda shp: pl.BlockSpec(shp, lambda i: (0, 0))
    oblk = pl.BlockSpec((RBP, 128), lambda i: (i, 0))
    return pl.pallas_call(
        _proj_body,
        grid=(GRID,),
        in_specs=[xblk, xblk, full((8, 256)), full((1, 256)),
                  full((8, 256)), full((1, 256))],
        out_specs=[oblk] * 4,
        out_shape=[jax.ShapeDtypeStruct((NP4, 128), jnp.float32)] * 4,
    )(xu8, xi8, Bu, btu, Bi, bti)


def _sage_out(alo, ahi, cp, hlo, hhi, Am, Ar, bt):
    inv = 1.0 / jnp.maximum(cp[...], 1.0)
    mcat = jnp.concatenate([alo[...] * inv, ahi[...] * inv], axis=1)
    hcat = jnp.concatenate([hlo[...], hhi[...]], axis=1)
    return jnp.maximum(mcat @ Am[...] + hcat @ Ar[...] + bt[...], 0.0)


def _layer_body(ai_lo, ai_hi, ci, hi_lo, hi_hi, ami, ari, bti,
                au_lo, au_hi, cu, hu_lo, hu_hi, amu, aru, btu,
                ni_lo, ni_hi, nu_lo, nu_hi):
    oi = _sage_out(ai_lo, ai_hi, ci, hi_lo, hi_hi, ami, ari, bti)
    ou = _sage_out(au_lo, au_hi, cu, hu_lo, hu_hi, amu, aru, btu)
    ni_lo[...] = oi[:, :128]
    ni_hi[...] = oi[:, 128:]
    nu_lo[...] = ou[:, :128]
    nu_hi[...] = ou[:, 128:]


def _final_body(ai_lo, ai_hi, ci, hi_lo, hi_hi, ami, ari, bti,
                au_lo, au_hi, cu, hu_lo, hu_hi, amu, aru, btu,
                sum_u, sum_i):
    i = pl.program_id(0)
    oi = _sage_out(ai_lo, ai_hi, ci, hi_lo, hi_hi, ami, ari, bti)
    ou = _sage_out(au_lo, au_hi, cu, hu_lo, hu_hi, amu, aru, btu)
    row = lax.broadcasted_iota(jnp.int32, (RBP, 256), 0) + i * RBP
    valid = row < (N // 4)
    oi = jnp.where(valid, oi, 0.0)
    ou = jnp.where(valid, ou, 0.0)

    @pl.when(i == 0)
    def _():
        sum_u[...] = jnp.zeros_like(sum_u)
        sum_i[...] = jnp.zeros_like(sum_i)

    sum_u[...] += jnp.sum(ou, axis=0, keepdims=True)
    sum_i[...] += jnp.sum(oi, axis=0, keepdims=True)


def _layer_specs():
    blk = pl.BlockSpec((RBP, 128), lambda i: (i, 0))
    w = pl.BlockSpec((256, 256), lambda i: (0, 0))
    b = pl.BlockSpec((1, 256), lambda i: (0, 0))
    return [blk, blk, blk, blk, blk, w, w, b,
            blk, blk, blk, blk, blk, w, w, b]


def _tc_layer(ai, ci, hi, wi, au, cu, hu, wu):
    oblk = pl.BlockSpec((RBP, 128), lambda i: (i, 0))
    return pl.pallas_call(
        _layer_body,
        grid=(GRID,),
        in_specs=_layer_specs(),
        out_specs=[oblk] * 4,
        out_shape=[jax.ShapeDtypeStruct((NP4, 128), jnp.float32)] * 4,
    )(ai[0], ai[1], ci, hi[0], hi[1], *wi,
      au[0], au[1], cu, hu[0], hu[1], *wu)


def _tc_final(ai, ci, hi, wi, au, cu, hu, wu):
    sblk = pl.BlockSpec((1, 256), lambda i: (0, 0))
    return pl.pallas_call(
        _final_body,
        grid=(GRID,),
        in_specs=_layer_specs(),
        out_specs=[sblk, sblk],
        out_shape=[jax.ShapeDtypeStruct((1, 256), jnp.float32)] * 2,
    )(ai[0], ai[1], ci, hi[0], hi[1], *wi,
      au[0], au[1], cu, hu[0], hu[1], *wu)


def _head_body(su, si, w1, b1, w2, b2, out):
    def fold(s):
        lo = s[:, 0:32] + s[:, 32:64] + s[:, 64:96] + s[:, 96:128]
        hi = s[:, 128:160] + s[:, 160:192] + s[:, 192:224] + s[:, 224:256]
        return jnp.concatenate([lo, hi], axis=1)

    pooled = jnp.concatenate([fold(su[...]), fold(si[...])], axis=1)
    hid = jnp.maximum(pooled @ w1[...] + b1[...], 0.0)
    out[...] = hid @ w2[...] + b2[...]


def _tc_head(sum_u, sum_i, Wh1, bh1, Wh2, bh2):
    return pl.pallas_call(
        _head_body,
        out_shape=jax.ShapeDtypeStruct((1, OUT), jnp.float32),
    )(sum_u, sum_i, Wh1, bh1.reshape(1, H), Wh2, bh2.reshape(1, OUT))


def _packed(a):
    return jnp.reshape(a, (NP4, 128))


def _unpacked(a):
    return jnp.reshape(a, (N_PAD, HH))


def kernel(x_user, x_item, W_in_user, b_in_user, W_in_item, b_in_item,
           miss_user, miss_item,
           Wl0_ri, bl0_ri, Wr0_ri, Wl0_ru, bl0_ru, Wr0_ru,
           Wl1_ri, bl1_ri, Wr1_ri, Wl1_ru, bl1_ru, Wr1_ru,
           Wh1, bh1, Wh2, bh2,
           ei_rates, ei_rated_by):
    pad = (N + (jnp.arange(E_PAD, dtype=jnp.int32) % (N_PAD - N))).astype(jnp.int32)

    def edges(ei):
        return (jnp.concatenate([ei[0], pad]).reshape(ROWS2D, LW),
                jnp.concatenate([ei[1], pad]).reshape(ROWS2D, LW))

    src_ri, dst_ri = edges(ei_rates)
    src_ru, dst_ru = edges(ei_rated_by)

    z32 = jnp.zeros((N_PAD, HH), jnp.float32)
    ones32 = jnp.ones((LW, HH), jnp.float32)

    def proj_w(Win, miss):
        Weff = jnp.stack([Win[0], Win[1] + miss[0]])
        Wb = Weff.reshape(2, 2, 32)
        eye4 = jnp.eye(4, dtype=Win.dtype)
        return jnp.einsum("ab,dpj->adpbj", eye4, Wb).reshape(8, 256)

    xu8 = jnp.pad(x_user, ((0, N_PAD - N), (0, 0))).reshape(NP4, 8)
    xi8 = jnp.pad(x_item, ((0, N_PAD - N), (0, 0))).reshape(NP4, 8)
    hu = _tc_proj(xu8, xi8,
                  proj_w(W_in_user, miss_user), _tile_bias(b_in_user),
                  proj_w(W_in_item, miss_item), _tile_bias(b_in_item))
    h_u, h_i = (hu[0], hu[1]), (hu[2], hu[3])

    cnt_i, cnt_u = _sc_cnt(dst_ri, dst_ru, ones32, z32)
    cnt_i, cnt_u = _packed(cnt_i), _packed(cnt_u)

    w0i = (_expand_w(Wl0_ri), _expand_w(Wr0_ri), _tile_bias(bl0_ri))
    w0u = (_expand_w(Wl0_ru), _expand_w(Wr0_ru), _tile_bias(bl0_ru))
    w1i = (_expand_w(Wl1_ri), _expand_w(Wr1_ri), _tile_bias(bl1_ri))
    w1u = (_expand_w(Wl1_ru), _expand_w(Wr1_ru), _tile_bias(bl1_ru))

    def mp(h_src, src2d, dst2d):
        o = _sc_mp(_unpacked(h_src[0]), _unpacked(h_src[1]), src2d, dst2d, z32)
        return _packed(o[0]), _packed(o[1])

    agg_i = mp(h_u, src_ri, dst_ri)
    agg_u = mp(h_i, src_ru, dst_ru)
    nh = _tc_layer(agg_i, cnt_i, h_i, w0i, agg_u, cnt_u, h_u, w0u)
    h_i, h_u = (nh[0], nh[1]), (nh[2], nh[3])

    agg_i = mp(h_u, src_ri, dst_ri)
    agg_u = mp(h_i, src_ru, dst_ru)
    sum_u, sum_i = _tc_final(agg_i, cnt_i, h_i, w1i, agg_u, cnt_u, h_u, w1u)

    return _tc_head(sum_u, sum_i, Wh1, bh1, Wh2, bh2)


# speedup vs baseline: 18.6058x; 1.0831x over previous
import jax
import jax.numpy as jnp
from jax import lax
from jax.experimental import pallas as pl
from jax.experimental.pallas import tpu as pltpu
from jax.experimental.pallas import tpu_sc as plsc

N = 50000
E = 800000
H = 64
HH = 32
OUT = 10
NS = 16
LW = 128
EROWS = E // LW
RPS = EROWS // NS
CH = 5
NCH_BASE = RPS // CH
NCH_LAST = (EROWS - (NS - 1) * RPS) // CH
N_PAD = 50048
NODE_SLICE = N_PAD // NS
NP4 = N_PAD // 4

_mesh = plsc.VectorSubcoreMesh(core_axis_name="c", subcore_axis_name="s")


def _sc_mp_body(table_lo, table_hi, ei3, zeros,
                out_lo, out_hi, sidxA, didxA, sidxB, didxB, rbuf, acc,
                gsem, ssem, isem):
    c = lax.axis_index("c")
    s = lax.axis_index("s")
    r0 = s * NODE_SLICE
    base = s * RPS
    nch = jnp.where(s == NS - 1, NCH_LAST, NCH_BASE)
    pltpu.sync_copy(zeros.at[pl.ds(r0, NODE_SLICE)], acc.at[pl.ds(r0, NODE_SLICE)])
    plsc.subcore_barrier()

    def main(table):
        def stage(k, sp, dp, sem):
            rb = base + k * CH
            pltpu.async_copy(ei3.at[0, pl.ds(rb, CH)], sp, sem)
            pltpu.async_copy(ei3.at[1, pl.ds(rb, CH)], dp, sem)

        def process(k, par, is_first):
            sp, dp = (sidxA, didxA) if par == 0 else (sidxB, didxB)
            nsp, ndp = (sidxB, didxB) if par == 0 else (sidxA, didxA)

            if not is_first:
                pltpu.make_async_copy(ei3.at[0, pl.ds(0, CH)], sp,
                                      isem.at[par]).wait()
                pltpu.make_async_copy(ei3.at[1, pl.ds(0, CH)], dp,
                                      isem.at[par]).wait()
            for j in range(CH):
                if not is_first:
                    pltpu.make_async_copy(rbuf.at[j], acc.at[dp.at[j]],
                                          ssem.at[j]).wait()
                pltpu.async_copy(table.at[sp.at[j]], rbuf.at[j], gsem.at[j])

            @pl.when(k + 1 < nch)
            def _():
                stage(k + 1, nsp, ndp, isem.at[1 - par])

            for j in range(CH):
                pltpu.make_async_copy(table.at[sp.at[j]], rbuf.at[j],
                                      gsem.at[j]).wait()
                pltpu.async_copy(rbuf.at[j], acc.at[dp.at[j]], ssem.at[j],
                                 add=True)

        stage(0, sidxA, didxA, isem.at[0])
        pltpu.make_async_copy(ei3.at[0, pl.ds(0, CH)], sidxA, isem.at[0]).wait()
        pltpu.make_async_copy(ei3.at[1, pl.ds(0, CH)], didxA, isem.at[0]).wait()
        process(0, 0, True)

        def pair(g, carry):
            process(2 * g + 1, 1, False)
            process(2 * g + 2, 0, False)
            return carry

        lax.fori_loop(0, (nch - 1) // 2, pair, 0)
        process(nch - 1, 1, False)
        for j in range(CH):
            pltpu.make_async_copy(rbuf.at[j], acc.at[didxB.at[j]],
                                  ssem.at[j]).wait()

    @pl.when(c == 0)
    def _():
        main(table_lo)

    @pl.when(c == 1)
    def _():
        main(table_hi)

    plsc.subcore_barrier()

    @pl.when(c == 0)
    def _():
        pltpu.sync_copy(acc.at[pl.ds(r0, NODE_SLICE)], out_lo.at[pl.ds(r0, NODE_SLICE)])

    @pl.when(c == 1)
    def _():
        pltpu.sync_copy(acc.at[pl.ds(r0, NODE_SLICE)], out_hi.at[pl.ds(r0, NODE_SLICE)])


_sc_mp = pl.kernel(
    _sc_mp_body,
    out_type=[jax.ShapeDtypeStruct((N_PAD, HH), jnp.float32)] * 2,
    mesh=_mesh,
    scratch_types=[
        pltpu.VMEM((CH, LW), jnp.int32),
        pltpu.VMEM((CH, LW), jnp.int32),
        pltpu.VMEM((CH, LW), jnp.int32),
        pltpu.VMEM((CH, LW), jnp.int32),
        pltpu.VMEM((CH, LW, HH), jnp.float32),
        pltpu.VMEM_SHARED((N_PAD, HH), jnp.float32),
        pltpu.SemaphoreType.DMA((CH,)),
        pltpu.SemaphoreType.DMA((CH,)),
        pltpu.SemaphoreType.DMA((2,)),
    ],
    compiler_params=pltpu.CompilerParams(use_tc_tiling_on_sc=False),
)


def _sc_cnt_body(ei3_a, ei3_b, ones_hbm, zeros,
                 out_a, out_b, didxA, didxB, ones_v, acc, osem, ssem, isem):
    c = lax.axis_index("c")
    s = lax.axis_index("s")
    r0 = s * NODE_SLICE
    base = s * RPS
    nch = jnp.where(s == NS - 1, NCH_LAST, NCH_BASE)
    pltpu.async_copy(ones_hbm, ones_v, osem).wait()
    pltpu.sync_copy(zeros.at[pl.ds(r0, NODE_SLICE)], acc.at[pl.ds(r0, NODE_SLICE)])
    plsc.subcore_barrier()

    def main(ei3):
        def stage(k, dp, sem):
            rb = base + k * CH
            pltpu.async_copy(ei3.at[1, pl.ds(rb, CH)], dp, sem)

        def process(k, par, is_first):
            dp = didxA if par == 0 else didxB
            ndp = didxB if par == 0 else didxA
            if not is_first:
                pltpu.make_async_copy(ei3.at[1, pl.ds(0, CH)], dp,
                                      isem.at[par]).wait()
            for j in range(CH):
                pltpu.async_copy(ones_v, acc.at[dp.at[j]],
                                 ssem.at[par * CH + j], add=True)
            if not is_first:
                for j in range(CH):
                    pltpu.make_async_copy(ones_v, acc.at[ndp.at[j]],
                                          ssem.at[(1 - par) * CH + j]).wait()

            @pl.when(k + 1 < nch)
            def _():
                stage(k + 1, ndp, isem.at[1 - par])

        stage(0, didxA, isem.at[0])
        pltpu.make_async_copy(ei3.at[1, pl.ds(0, CH)], didxA, isem.at[0]).wait()
        process(0, 0, True)

        def pair(g, carry):
            process(2 * g + 1, 1, False)
            process(2 * g + 2, 0, False)
            return carry

        lax.fori_loop(0, (nch - 1) // 2, pair, 0)
        process(nch - 1, 1, False)
        for j in range(CH):
            pltpu.make_async_copy(ones_v, acc.at[didxB.at[j]],
                                  ssem.at[CH + j]).wait()

    @pl.when(c == 0)
    def _():
        main(ei3_a)

    @pl.when(c == 1)
    def _():
        main(ei3_b)

    plsc.subcore_barrier()

    @pl.when(c == 0)
    def _():
        pltpu.sync_copy(acc.at[pl.ds(r0, NODE_SLICE)], out_a.at[pl.ds(r0, NODE_SLICE)])

    @pl.when(c == 1)
    def _():
        pltpu.sync_copy(acc.at[pl.ds(r0, NODE_SLICE)], out_b.at[pl.ds(r0, NODE_SLICE)])


_sc_cnt = pl.kernel(
    _sc_cnt_body,
    out_type=[jax.ShapeDtypeStruct((N_PAD, HH), jnp.float32)] * 2,
    mesh=_mesh,
    scratch_types=[
        pltpu.VMEM((CH, LW), jnp.int32),
        pltpu.VMEM((CH, LW), jnp.int32),
        pltpu.VMEM((LW, HH), jnp.float32),
        pltpu.VMEM_SHARED((N_PAD, HH), jnp.float32),
        pltpu.SemaphoreType.DMA,
        pltpu.SemaphoreType.DMA((2 * CH,)),
        pltpu.SemaphoreType.DMA((2,)),
    ],
    compiler_params=pltpu.CompilerParams(use_tc_tiling_on_sc=False),
)


RBP = 512
GRID = 25


def _expand_w(Wfull):
    Wb = Wfull.reshape(2, 32, 2, 32)
    eye4 = jnp.eye(4, dtype=Wfull.dtype)
    return jnp.einsum("ab,pkqj->pakqbj", eye4, Wb).reshape(256, 256)


def _tile_bias(b):
    return jnp.concatenate(
        [jnp.tile(b[:HH], 4), jnp.tile(b[HH:], 4)]).reshape(1, 256)


def _proj_body(xu_ref, xi_ref, bu_ref, btu_ref, bi_ref, bti_ref,
               hu_lo, hu_hi, hi_lo, hi_hi):
    def proj(x, Bw, bt):
        h = jnp.maximum(x[...] @ Bw[...] + bt[...], 0.0)
        return h[:, :128], h[:, 128:]

    hu_lo[...], hu_hi[...] = proj(xu_ref, bu_ref, btu_ref)
    hi_lo[...], hi_hi[...] = proj(xi_ref, bi_ref, bti_ref)


def _tc_proj(xu8, xi8, Bu, btu, Bi, bti):
    xblk = pl.BlockSpec((RBP, 8), lambda i: (i, 0))
    full = lambda shp: pl.BlockSpec(shp, lambda i: (0, 0))
    oblk = pl.BlockSpec((RBP, 128), lambda i: (i, 0))
    return pl.pallas_call(
        _proj_body,
        grid=(GRID,),
        in_specs=[xblk, xblk, full((8, 256)), full((1, 256)),
                  full((8, 256)), full((1, 256))],
        out_specs=[oblk] * 4,
        out_shape=[jax.ShapeDtypeStruct((NP4, 128), jnp.float32)] * 4,
    )(xu8, xi8, Bu, btu, Bi, bti)


def _sage_out(alo, ahi, cp, hlo, hhi, Am, Ar, bt):
    inv = 1.0 / jnp.maximum(cp[...], 1.0)
    mcat = jnp.concatenate([alo[...] * inv, ahi[...] * inv], axis=1)
    hcat = jnp.concatenate([hlo[...], hhi[...]], axis=1)
    return jnp.maximum(mcat @ Am[...] + hcat @ Ar[...] + bt[...], 0.0)


def _layer_body(ai_lo, ai_hi, ci, hi_lo, hi_hi, ami, ari, bti,
                au_lo, au_hi, cu, hu_lo, hu_hi, amu, aru, btu,
                ni_lo, ni_hi, nu_lo, nu_hi):
    oi = _sage_out(ai_lo, ai_hi, ci, hi_lo, hi_hi, ami, ari, bti)
    ou = _sage_out(au_lo, au_hi, cu, hu_lo, hu_hi, amu, aru, btu)
    ni_lo[...] = oi[:, :128]
    ni_hi[...] = oi[:, 128:]
    nu_lo[...] = ou[:, :128]
    nu_hi[...] = ou[:, 128:]


def _final_body(ai_lo, ai_hi, ci, hi_lo, hi_hi, ami, ari, bti,
                au_lo, au_hi, cu, hu_lo, hu_hi, amu, aru, btu,
                sum_u, sum_i):
    i = pl.program_id(0)
    oi = _sage_out(ai_lo, ai_hi, ci, hi_lo, hi_hi, ami, ari, bti)
    ou = _sage_out(au_lo, au_hi, cu, hu_lo, hu_hi, amu, aru, btu)
    row = lax.broadcasted_iota(jnp.int32, (RBP, 256), 0) + i * RBP
    valid = row < (N // 4)
    oi = jnp.where(valid, oi, 0.0)
    ou = jnp.where(valid, ou, 0.0)

    @pl.when(i == 0)
    def _():
        sum_u[...] = jnp.zeros_like(sum_u)
        sum_i[...] = jnp.zeros_like(sum_i)

    sum_u[...] += jnp.sum(ou, axis=0, keepdims=True)
    sum_i[...] += jnp.sum(oi, axis=0, keepdims=True)


def _layer_specs():
    blk = pl.BlockSpec((RBP, 128), lambda i: (i, 0))
    w = pl.BlockSpec((256, 256), lambda i: (0, 0))
    b = pl.BlockSpec((1, 256), lambda i: (0, 0))
    return [blk, blk, blk, blk, blk, w, w, b,
            blk, blk, blk, blk, blk, w, w, b]


def _tc_layer(ai, ci, hi, wi, au, cu, hu, wu):
    oblk = pl.BlockSpec((RBP, 128), lambda i: (i, 0))
    return pl.pallas_call(
        _layer_body,
        grid=(GRID,),
        in_specs=_layer_specs(),
        out_specs=[oblk] * 4,
        out_shape=[jax.ShapeDtypeStruct((NP4, 128), jnp.float32)] * 4,
    )(ai[0], ai[1], ci, hi[0], hi[1], *wi,
      au[0], au[1], cu, hu[0], hu[1], *wu)


def _tc_final(ai, ci, hi, wi, au, cu, hu, wu):
    sblk = pl.BlockSpec((1, 256), lambda i: (0, 0))
    return pl.pallas_call(
        _final_body,
        grid=(GRID,),
        in_specs=_layer_specs(),
        out_specs=[sblk, sblk],
        out_shape=[jax.ShapeDtypeStruct((1, 256), jnp.float32)] * 2,
    )(ai[0], ai[1], ci, hi[0], hi[1], *wi,
      au[0], au[1], cu, hu[0], hu[1], *wu)


def _head_body(su, si, w1, b1, w2, b2, out):
    def fold(s):
        lo = s[:, 0:32] + s[:, 32:64] + s[:, 64:96] + s[:, 96:128]
        hi = s[:, 128:160] + s[:, 160:192] + s[:, 192:224] + s[:, 224:256]
        return jnp.concatenate([lo, hi], axis=1)

    pooled = jnp.concatenate([fold(su[...]), fold(si[...])], axis=1)
    hid = jnp.maximum(pooled @ w1[...] + b1[...], 0.0)
    out[...] = hid @ w2[...] + b2[...]


def _tc_head(sum_u, sum_i, Wh1, bh1, Wh2, bh2):
    return pl.pallas_call(
        _head_body,
        out_shape=jax.ShapeDtypeStruct((1, OUT), jnp.float32),
    )(sum_u, sum_i, Wh1, bh1.reshape(1, H), Wh2, bh2.reshape(1, OUT))


def _packed(a):
    return jnp.reshape(a, (NP4, 128))


def _unpacked(a):
    return jnp.reshape(a, (N_PAD, HH))


def kernel(x_user, x_item, W_in_user, b_in_user, W_in_item, b_in_item,
           miss_user, miss_item,
           Wl0_ri, bl0_ri, Wr0_ri, Wl0_ru, bl0_ru, Wr0_ru,
           Wl1_ri, bl1_ri, Wr1_ri, Wl1_ru, bl1_ru, Wr1_ru,
           Wh1, bh1, Wh2, bh2,
           ei_rates, ei_rated_by):
    ei3_ri = ei_rates.reshape(2, EROWS, LW)
    ei3_ru = ei_rated_by.reshape(2, EROWS, LW)

    z32 = jnp.zeros((N_PAD, HH), jnp.float32)
    ones32 = jnp.ones((LW, HH), jnp.float32)

    def proj_w(Win, miss):
        Weff = jnp.stack([Win[0], Win[1] + miss[0]])
        Wb = Weff.reshape(2, 2, 32)
        eye4 = jnp.eye(4, dtype=Win.dtype)
        return jnp.einsum("ab,dpj->adpbj", eye4, Wb).reshape(8, 256)

    xu8 = jnp.pad(x_user, ((0, N_PAD - N), (0, 0))).reshape(NP4, 8)
    xi8 = jnp.pad(x_item, ((0, N_PAD - N), (0, 0))).reshape(NP4, 8)
    hu = _tc_proj(xu8, xi8,
                  proj_w(W_in_user, miss_user), _tile_bias(b_in_user),
                  proj_w(W_in_item, miss_item), _tile_bias(b_in_item))
    h_u, h_i = (hu[0], hu[1]), (hu[2], hu[3])

    cnt_i, cnt_u = _sc_cnt(ei3_ri, ei3_ru, ones32, z32)
    cnt_i, cnt_u = _packed(cnt_i), _packed(cnt_u)

    w0i = (_expand_w(Wl0_ri), _expand_w(Wr0_ri), _tile_bias(bl0_ri))
    w0u = (_expand_w(Wl0_ru), _expand_w(Wr0_ru), _tile_bias(bl0_ru))
    w1i = (_expand_w(Wl1_ri), _expand_w(Wr1_ri), _tile_bias(bl1_ri))
    w1u = (_expand_w(Wl1_ru), _expand_w(Wr1_ru), _tile_bias(bl1_ru))

    def mp(h_src, ei3):
        o = _sc_mp(_unpacked(h_src[0]), _unpacked(h_src[1]), ei3, z32)
        return _packed(o[0]), _packed(o[1])

    agg_i = mp(h_u, ei3_ri)
    agg_u = mp(h_i, ei3_ru)
    nh = _tc_layer(agg_i, cnt_i, h_i, w0i, agg_u, cnt_u, h_u, w0u)
    h_i, h_u = (nh[0], nh[1]), (nh[2], nh[3])

    agg_i = mp(h_u, ei3_ri)
    agg_u = mp(h_i, ei3_ru)
    sum_u, sum_i = _tc_final(agg_i, cnt_i, h_i, w1i, agg_u, cnt_u, h_u, w1u)

    return _tc_head(sum_u, sum_i, Wh1, bh1, Wh2, bh2)
